# baseline (speedup 1.0000x reference)
import functools

import jax
import jax.numpy as jnp
from jax import lax
from jax.experimental import pallas as pl
from jax.experimental.pallas import tpu as pltpu
from jax.experimental.pallas import tpu_sc as plsc

_B = 4
_L = 8192
_NC = 1
_NS = 16
_NW = _NC * _NS
_CHUNK = _B * _L // _NW
_HALF = _CHUNK // 2
_WPR = _L // _CHUNK
_LANES = 16
_UNROLL = 8

_mesh = plsc.VectorSubcoreMesh(core_axis_name="c", subcore_axis_name="s",
                               num_cores=1)


@functools.partial(
    pl.kernel,
    mesh=_mesh,
    out_type=jax.ShapeDtypeStruct((_B, _L), jnp.float32),
    scratch_types=[
        pltpu.VMEM((_CHUNK,), jnp.float32),
        pltpu.VMEM((_CHUNK,), jnp.float32),
        pltpu.SemaphoreType.DMA,
        pltpu.SemaphoreType.DMA,
        pltpu.SemaphoreType.DMA,
        pltpu.SemaphoreType.DMA,
        pltpu.SemaphoreType.DMA,
    ],
)
def _pos_add_sc(x_hbm, pos_hbm, out_hbm, x_v, pos_v, s1, s2, s3, s4, s5):
    wid = lax.axis_index("s") * _NC + lax.axis_index("c")
    row = wid // _WPR
    col = (wid % _WPR) * _CHUNK

    cp_x1 = pltpu.async_copy(x_hbm.at[row, pl.ds(col, _HALF)],
                             x_v.at[pl.ds(0, _HALF)], s1)
    cp_p1 = pltpu.async_copy(pos_hbm.at[pl.ds(col, _HALF)],
                             pos_v.at[pl.ds(0, _HALF)], s2)
    cp_x2 = pltpu.async_copy(x_hbm.at[row, pl.ds(col + _HALF, _HALF)],
                             x_v.at[pl.ds(_HALF, _HALF)], s3)
    cp_p2 = pltpu.async_copy(pos_hbm.at[pl.ds(col + _HALF, _HALF)],
                             pos_v.at[pl.ds(_HALF, _HALF)], s4)

    def add_half(lo):
        def body(i, _):
            base = lo + i * (_LANES * _UNROLL)
            for j in range(_UNROLL):
                sl = pl.ds(base + j * _LANES, _LANES)
                x_v[sl] = x_v[sl] + pos_v[sl]
            return _

        lax.fori_loop(0, _HALF // (_LANES * _UNROLL), body, None)

    cp_x1.wait()
    cp_p1.wait()
    add_half(0)
    cp_o1 = pltpu.async_copy(x_v.at[pl.ds(0, _HALF)],
                             out_hbm.at[row, pl.ds(col, _HALF)], s5)
    cp_x2.wait()
    cp_p2.wait()
    add_half(_HALF)
    pltpu.sync_copy(x_v.at[pl.ds(_HALF, _HALF)],
                    out_hbm.at[row, pl.ds(col + _HALF, _HALF)])
    cp_o1.wait()


def kernel(x, pos_table):
    return _pos_add_sc(x, pos_table.reshape(-1))


# speedup vs baseline: 1.0336x; 1.0336x over previous
import functools

import jax
import jax.numpy as jnp
from jax import lax
from jax.experimental import pallas as pl
from jax.experimental.pallas import tpu as pltpu
from jax.experimental.pallas import tpu_sc as plsc

_B = 4
_L = 8192
_NC = 1
_NS = 16
_NW = _NC * _NS
_CHUNK = _B * _L // _NW
_WPR = _L // _CHUNK
_LANES = 16

_mesh = plsc.VectorSubcoreMesh(core_axis_name="c", subcore_axis_name="s",
                               num_cores=1)


@functools.partial(
    pl.kernel,
    mesh=_mesh,
    out_type=jax.ShapeDtypeStruct((_B, _L), jnp.float32),
    scratch_types=[
        pltpu.VMEM((_CHUNK,), jnp.float32),
        pltpu.VMEM((_CHUNK,), jnp.float32),
        pltpu.SemaphoreType.DMA,
        pltpu.SemaphoreType.DMA,
    ],
)
def _pos_add_sc(x_hbm, pos_hbm, out_hbm, x_v, pos_v, sem_x, sem_p):
    wid = lax.axis_index("s") * _NC + lax.axis_index("c")
    row = wid // _WPR
    col = (wid % _WPR) * _CHUNK
    cp_x = pltpu.async_copy(x_hbm.at[row, pl.ds(col, _CHUNK)], x_v, sem_x)
    cp_p = pltpu.async_copy(pos_hbm.at[pl.ds(col, _CHUNK)], pos_v, sem_p)
    cp_x.wait()
    cp_p.wait()
    @plsc.parallel_loop(0, _CHUNK, _LANES, unroll=8)
    def _(i):
        sl = pl.ds(i, _LANES)
        x_v[sl] = x_v[sl] + pos_v[sl]
    pltpu.sync_copy(x_v, out_hbm.at[row, pl.ds(col, _CHUNK)])


def kernel(x, pos_table):
    return _pos_add_sc(x, pos_table.reshape(-1))
